# baseline (speedup 1.0000x reference)
import functools

import jax
import jax.numpy as jnp
from jax import lax
from jax.experimental import pallas as pl
from jax.experimental.pallas import tpu as pltpu
from jax.experimental.pallas import tpu_sc as plsc

N = 10000
E = 160000
D = 256
HALF = 128
N_LAYERS_K = 3
NS = 16
CHUNK = 128
EPAD = 163840
ED_PER_TILE = EPAD // NS
NCHUNK = ED_PER_TILE // CHUNK
EROWS = EPAD // CHUNK
NPAD = 10240
ROWS_PER_TILE = NPAD // NS

BN = 1000
NBLK = N // BN

IB = 16


def _sc_agg(x_stack, srcg, dst):
    mesh = plsc.VectorSubcoreMesh(core_axis_name="c", subcore_axis_name="s")

    @functools.partial(
        pl.kernel,
        out_type=jax.ShapeDtypeStruct((2 * NPAD, HALF), jnp.float32),
        mesh=mesh,
        scratch_types=[
            pltpu.VMEM_SHARED((NPAD, HALF), jnp.float32),
            pltpu.VMEM((IB, CHUNK), jnp.int32),
            pltpu.VMEM((IB, CHUNK), jnp.int32),
            pltpu.VMEM((CHUNK, HALF), jnp.float32),
        ],
    )
    def k(x_hbm, srcg_hbm, dst_hbm, out_hbm, acc, gidx, didx, rows):
        c = lax.axis_index("c")
        s = lax.axis_index("s")

        @pl.loop(0, CHUNK)
        def _(r):
            for cc in range(HALF // 16):
                rows[r, pl.ds(cc * 16, 16)] = jnp.zeros((16,), jnp.float32)

        @pl.loop(0, ROWS_PER_TILE, step=CHUNK)
        def _(r):
            pltpu.sync_copy(rows, acc.at[pl.ds(s * ROWS_PER_TILE + r, CHUNK)])

        plsc.subcore_barrier()

        @pl.loop(0, NCHUNK)
        def _(j):
            r = lax.rem(j, IB)

            @pl.when(r == 0)
            def _():
                base = pl.multiple_of(s * NCHUNK + j, IB)
                pltpu.sync_copy(srcg_hbm.at[pl.ds(c * EROWS + base, IB)], gidx)
                pltpu.sync_copy(dst_hbm.at[pl.ds(base, IB)], didx)

            pltpu.sync_copy(x_hbm.at[gidx.at[r]], rows)
            pltpu.sync_copy(rows, acc.at[didx.at[r]], add=True)

        plsc.subcore_barrier()
        out_base = c * NPAD + s * ROWS_PER_TILE
        pltpu.sync_copy(acc.at[pl.ds(s * ROWS_PER_TILE, ROWS_PER_TILE)],
                        out_hbm.at[pl.ds(out_base, ROWS_PER_TILE)])

    return k(x_stack, srcg, dst)


_SQRT_HALF = 0.7071067811865476
_INV_SQRT2 = 0.7071067811865476


def _gelu_exact(h):
    return 0.5 * h * (1.0 + lax.erf(h * _SQRT_HALF))


def _tc_mlp(x, agg_lo, agg_hi, W1, b1, W2, b2, eps_i):
    def body(x_ref, lo_ref, hi_ref, w1_ref, b1_ref, w2_ref, b2_ref, e_ref,
             z_ref, s_ref, ss_ref):
        i = pl.program_id(0)
        agg = jnp.concatenate([lo_ref[...], hi_ref[...]], axis=1)
        h0 = (1.0 + e_ref[0, 0]) * x_ref[...] + agg
        h1 = jnp.dot(h0, w1_ref[...], preferred_element_type=jnp.float32,
                     precision=lax.Precision.HIGHEST) + b1_ref[...]
        h1 = _gelu_exact(h1)
        z = jnp.dot(h1, w2_ref[...], preferred_element_type=jnp.float32,
                    precision=lax.Precision.HIGHEST) + b2_ref[...]
        z_ref[...] = z
        rowid = lax.broadcasted_iota(jnp.int32, (8, D), 0)
        pad_s = jnp.where(rowid == 0, jnp.sum(z, axis=0, keepdims=True), 0.0)
        pad_ss = jnp.where(rowid == 0, jnp.sum(z * z, axis=0, keepdims=True), 0.0)

        @pl.when(i == 0)
        def _():
            s_ref[...] = pad_s
            ss_ref[...] = pad_ss

        @pl.when(i > 0)
        def _():
            s_ref[...] = s_ref[...] + pad_s
            ss_ref[...] = ss_ref[...] + pad_ss

    return pl.pallas_call(
        body,
        grid=(NBLK,),
        in_specs=[
            pl.BlockSpec((BN, D), lambda i: (i, 0)),
            pl.BlockSpec((BN, HALF), lambda i: (i, 0)),
            pl.BlockSpec((BN, HALF), lambda i: (i, 0)),
            pl.BlockSpec((D, D), lambda i: (0, 0)),
            pl.BlockSpec((1, D), lambda i: (0, 0)),
            pl.BlockSpec((D, D), lambda i: (0, 0)),
            pl.BlockSpec((1, D), lambda i: (0, 0)),
            pl.BlockSpec((1, 1), lambda i: (0, 0)),
        ],
        out_specs=[
            pl.BlockSpec((BN, D), lambda i: (i, 0)),
            pl.BlockSpec((8, D), lambda i: (0, 0)),
            pl.BlockSpec((8, D), lambda i: (0, 0)),
        ],
        out_shape=[
            jax.ShapeDtypeStruct((N, D), jnp.float32),
            jax.ShapeDtypeStruct((8, D), jnp.float32),
            jax.ShapeDtypeStruct((8, D), jnp.float32),
        ],
    )(x, agg_lo, agg_hi, W1, b1.reshape(1, D), W2, b2.reshape(1, D),
      eps_i.reshape(1, 1))


def _tc_norm(z, x, ssum, ssq, gamma_i, beta_i):
    def body(z_ref, x_ref, s_ref, ss_ref, g_ref, b_ref, o_ref):
        ssum_v = jnp.sum(s_ref[...], axis=0, keepdims=True)
        ssq_v = jnp.sum(ss_ref[...], axis=0, keepdims=True)
        mean = ssum_v * (1.0 / N)
        var = ssq_v * (1.0 / N) - mean * mean
        inv = lax.rsqrt(var + 1e-5)
        h = (z_ref[...] - mean) * (inv * g_ref[...]) + b_ref[...]
        h = _gelu_exact(h)
        o_ref[...] = (x_ref[...] + h) * _INV_SQRT2

    return pl.pallas_call(
        body,
        grid=(NBLK,),
        in_specs=[
            pl.BlockSpec((BN, D), lambda i: (i, 0)),
            pl.BlockSpec((BN, D), lambda i: (i, 0)),
            pl.BlockSpec((8, D), lambda i: (0, 0)),
            pl.BlockSpec((8, D), lambda i: (0, 0)),
            pl.BlockSpec((1, D), lambda i: (0, 0)),
            pl.BlockSpec((1, D), lambda i: (0, 0)),
        ],
        out_specs=pl.BlockSpec((BN, D), lambda i: (i, 0)),
        out_shape=jax.ShapeDtypeStruct((N, D), jnp.float32),
    )(z, x, ssum, ssq, gamma_i.reshape(1, D), beta_i.reshape(1, D))


def kernel(x, edge_index, edge_attr, W1s, b1s, W2s, b2s, eps, gamma, beta):
    pad = EPAD - E
    src = jnp.concatenate(
        [edge_index[0].astype(jnp.int32), jnp.zeros((pad,), jnp.int32)]
    ).reshape(EROWS, CHUNK)
    srcg = jnp.concatenate([src, src + N], axis=0)
    dst = jnp.concatenate(
        [edge_index[1].astype(jnp.int32), jnp.full((pad,), N, jnp.int32)]
    ).reshape(EROWS, CHUNK)

    for i in range(N_LAYERS_K):
        x_stack = jnp.concatenate([x[:, :HALF], x[:, HALF:]], axis=0)
        agg2 = _sc_agg(x_stack, srcg, dst)
        agg_lo = agg2[:N]
        agg_hi = agg2[NPAD:NPAD + N]
        z, ssum, ssq = _tc_mlp(x, agg_lo, agg_hi, W1s[i], b1s[i], W2s[i],
                               b2s[i], eps[i])
        x = _tc_norm(z, x, ssum, ssq, gamma[i], beta[i])
    return x


# speedup vs baseline: 1.0607x; 1.0607x over previous
import functools

import jax
import jax.numpy as jnp
from jax import lax
from jax.experimental import pallas as pl
from jax.experimental.pallas import tpu as pltpu
from jax.experimental.pallas import tpu_sc as plsc

N = 10000
E = 160000
D = 256
HALF = 128
N_LAYERS_K = 3
NS = 16
CHUNK = 128
EPAD = 163840
ED_PER_TILE = EPAD // NS
NCHUNK = ED_PER_TILE // CHUNK
EROWS = EPAD // CHUNK
NPAD = 10240
ROWS_PER_TILE = NPAD // NS

BN = 1000
NBLK = N // BN

IB = 16


def _sc_agg(x_stack, srcg, dst):
    mesh = plsc.VectorSubcoreMesh(core_axis_name="c", subcore_axis_name="s")

    @functools.partial(
        pl.kernel,
        out_type=jax.ShapeDtypeStruct((2 * NPAD, HALF), jnp.float32),
        mesh=mesh,
        scratch_types=[
            pltpu.VMEM_SHARED((NPAD, HALF), jnp.float32),
            pltpu.VMEM((IB, CHUNK), jnp.int32),
            pltpu.VMEM((IB, CHUNK), jnp.int32),
            pltpu.VMEM((CHUNK, HALF), jnp.float32),
            pltpu.VMEM((CHUNK, HALF), jnp.float32),
            pltpu.SemaphoreType.DMA,
            pltpu.SemaphoreType.DMA,
        ],
    )
    def k(x_hbm, srcg_hbm, dst_hbm, out_hbm, acc, gidx, didx, rowsA, rowsB,
          semA, semB):
        c = lax.axis_index("c")
        s = lax.axis_index("s")

        @pl.loop(0, CHUNK)
        def _(r):
            for cc in range(HALF // 16):
                rowsA[r, pl.ds(cc * 16, 16)] = jnp.zeros((16,), jnp.float32)

        @pl.loop(0, ROWS_PER_TILE, step=CHUNK)
        def _(r):
            pltpu.sync_copy(rowsA, acc.at[pl.ds(s * ROWS_PER_TILE + r, CHUNK)])

        plsc.subcore_barrier()

        @pl.loop(0, NCHUNK // IB)
        def _(b):
            base = pl.multiple_of(s * NCHUNK, IB) + b * IB
            pltpu.sync_copy(srcg_hbm.at[pl.ds(c * EROWS + base, IB)], gidx)
            pltpu.sync_copy(dst_hbm.at[pl.ds(base, IB)], didx)

            @pl.loop(0, IB, step=2)
            def _(r):
                hA = pltpu.async_copy(x_hbm.at[gidx.at[r]], rowsA, semA)
                hB = pltpu.async_copy(x_hbm.at[gidx.at[r + 1]], rowsB, semB)
                hA.wait()
                pltpu.sync_copy(rowsA, acc.at[didx.at[r]], add=True)
                hB.wait()
                pltpu.sync_copy(rowsB, acc.at[didx.at[r + 1]], add=True)

        plsc.subcore_barrier()
        out_base = c * NPAD + s * ROWS_PER_TILE
        pltpu.sync_copy(acc.at[pl.ds(s * ROWS_PER_TILE, ROWS_PER_TILE)],
                        out_hbm.at[pl.ds(out_base, ROWS_PER_TILE)])

    return k(x_stack, srcg, dst)


_SQRT_HALF = 0.7071067811865476
_INV_SQRT2 = 0.7071067811865476


def _gelu_exact(h):
    return 0.5 * h * (1.0 + lax.erf(h * _SQRT_HALF))


def _tc_mlp(x, agg_lo, agg_hi, W1, b1, W2, b2, eps_i):
    def body(x_ref, lo_ref, hi_ref, w1_ref, b1_ref, w2_ref, b2_ref, e_ref,
             z_ref, s_ref, ss_ref):
        i = pl.program_id(0)
        agg = jnp.concatenate([lo_ref[...], hi_ref[...]], axis=1)
        h0 = (1.0 + e_ref[0, 0]) * x_ref[...] + agg
        h1 = jnp.dot(h0, w1_ref[...], preferred_element_type=jnp.float32,
                     precision=lax.Precision.HIGHEST) + b1_ref[...]
        h1 = _gelu_exact(h1)
        z = jnp.dot(h1, w2_ref[...], preferred_element_type=jnp.float32,
                    precision=lax.Precision.HIGHEST) + b2_ref[...]
        z_ref[...] = z
        rowid = lax.broadcasted_iota(jnp.int32, (8, D), 0)
        pad_s = jnp.where(rowid == 0, jnp.sum(z, axis=0, keepdims=True), 0.0)
        pad_ss = jnp.where(rowid == 0, jnp.sum(z * z, axis=0, keepdims=True), 0.0)

        @pl.when(i == 0)
        def _():
            s_ref[...] = pad_s
            ss_ref[...] = pad_ss

        @pl.when(i > 0)
        def _():
            s_ref[...] = s_ref[...] + pad_s
            ss_ref[...] = ss_ref[...] + pad_ss

    return pl.pallas_call(
        body,
        grid=(NBLK,),
        in_specs=[
            pl.BlockSpec((BN, D), lambda i: (i, 0)),
            pl.BlockSpec((BN, HALF), lambda i: (i, 0)),
            pl.BlockSpec((BN, HALF), lambda i: (i, 0)),
            pl.BlockSpec((D, D), lambda i: (0, 0)),
            pl.BlockSpec((1, D), lambda i: (0, 0)),
            pl.BlockSpec((D, D), lambda i: (0, 0)),
            pl.BlockSpec((1, D), lambda i: (0, 0)),
            pl.BlockSpec((1, 1), lambda i: (0, 0)),
        ],
        out_specs=[
            pl.BlockSpec((BN, D), lambda i: (i, 0)),
            pl.BlockSpec((8, D), lambda i: (0, 0)),
            pl.BlockSpec((8, D), lambda i: (0, 0)),
        ],
        out_shape=[
            jax.ShapeDtypeStruct((N, D), jnp.float32),
            jax.ShapeDtypeStruct((8, D), jnp.float32),
            jax.ShapeDtypeStruct((8, D), jnp.float32),
        ],
    )(x, agg_lo, agg_hi, W1, b1.reshape(1, D), W2, b2.reshape(1, D),
      eps_i.reshape(1, 1))


def _tc_norm(z, x, ssum, ssq, gamma_i, beta_i):
    def body(z_ref, x_ref, s_ref, ss_ref, g_ref, b_ref, o_ref):
        ssum_v = jnp.sum(s_ref[...], axis=0, keepdims=True)
        ssq_v = jnp.sum(ss_ref[...], axis=0, keepdims=True)
        mean = ssum_v * (1.0 / N)
        var = ssq_v * (1.0 / N) - mean * mean
        inv = lax.rsqrt(var + 1e-5)
        h = (z_ref[...] - mean) * (inv * g_ref[...]) + b_ref[...]
        h = _gelu_exact(h)
        o_ref[...] = (x_ref[...] + h) * _INV_SQRT2

    return pl.pallas_call(
        body,
        grid=(NBLK,),
        in_specs=[
            pl.BlockSpec((BN, D), lambda i: (i, 0)),
            pl.BlockSpec((BN, D), lambda i: (i, 0)),
            pl.BlockSpec((8, D), lambda i: (0, 0)),
            pl.BlockSpec((8, D), lambda i: (0, 0)),
            pl.BlockSpec((1, D), lambda i: (0, 0)),
            pl.BlockSpec((1, D), lambda i: (0, 0)),
        ],
        out_specs=pl.BlockSpec((BN, D), lambda i: (i, 0)),
        out_shape=jax.ShapeDtypeStruct((N, D), jnp.float32),
    )(z, x, ssum, ssq, gamma_i.reshape(1, D), beta_i.reshape(1, D))


def kernel(x, edge_index, edge_attr, W1s, b1s, W2s, b2s, eps, gamma, beta):
    pad = EPAD - E
    src = jnp.concatenate(
        [edge_index[0].astype(jnp.int32), jnp.zeros((pad,), jnp.int32)]
    ).reshape(EROWS, CHUNK)
    srcg = jnp.concatenate([src, src + N], axis=0)
    dst = jnp.concatenate(
        [edge_index[1].astype(jnp.int32), jnp.full((pad,), N, jnp.int32)]
    ).reshape(EROWS, CHUNK)

    for i in range(N_LAYERS_K):
        x_stack = jnp.concatenate([x[:, :HALF], x[:, HALF:]], axis=0)
        agg2 = _sc_agg(x_stack, srcg, dst)
        agg_lo = agg2[:N]
        agg_hi = agg2[NPAD:NPAD + N]
        z, ssum, ssq = _tc_mlp(x, agg_lo, agg_hi, W1s[i], b1s[i], W2s[i],
                               b2s[i], eps[i])
        x = _tc_norm(z, x, ssum, ssq, gamma[i], beta[i])
    return x


# speedup vs baseline: 1.1146x; 1.0508x over previous
import functools

import jax
import jax.numpy as jnp
from jax import lax
from jax.experimental import pallas as pl
from jax.experimental.pallas import tpu as pltpu
from jax.experimental.pallas import tpu_sc as plsc

N = 10000
E = 160000
D = 256
HALF = 128
N_LAYERS_K = 3
NS = 16
CHUNK = 128
EPAD = 163840
ED_PER_TILE = EPAD // NS
NCHUNK = ED_PER_TILE // CHUNK
EROWS = EPAD // CHUNK
NPAD = 10240
ROWS_PER_TILE = NPAD // NS

BN = 1000
NBLK = N // BN

IB = 8


def _sc_agg(x_stack, srcg, dst):
    mesh = plsc.VectorSubcoreMesh(core_axis_name="c", subcore_axis_name="s")

    @functools.partial(
        pl.kernel,
        out_type=jax.ShapeDtypeStruct((2 * NPAD, HALF), jnp.float32),
        mesh=mesh,
        scratch_types=[
            pltpu.VMEM_SHARED((NPAD, HALF), jnp.float32),
            pltpu.VMEM((IB, CHUNK), jnp.int32),
            pltpu.VMEM((IB, CHUNK), jnp.int32),
            pltpu.VMEM((CHUNK, HALF), jnp.float32),
            pltpu.VMEM((CHUNK, HALF), jnp.float32),
            pltpu.SemaphoreType.DMA,
            pltpu.SemaphoreType.DMA,
            pltpu.SemaphoreType.DMA,
            pltpu.SemaphoreType.DMA,
        ],
    )
    def k(x_hbm, srcg_hbm, dst_hbm, out_hbm, acc, gidx, didx, rowsA, rowsB,
          semGA, semGB, semSA, semSB):
        c = lax.axis_index("c")
        s = lax.axis_index("s")

        @pl.loop(0, CHUNK)
        def _(r):
            for cc in range(HALF // 16):
                rowsA[r, pl.ds(cc * 16, 16)] = jnp.zeros((16,), jnp.float32)

        @pl.loop(0, ROWS_PER_TILE, step=CHUNK)
        def _(r):
            pltpu.sync_copy(rowsA, acc.at[pl.ds(s * ROWS_PER_TILE + r, CHUNK)])

        plsc.subcore_barrier()

        @pl.loop(0, NCHUNK // IB)
        def _(b):
            base = pl.multiple_of(s * NCHUNK, IB) + b * IB
            pltpu.sync_copy(srcg_hbm.at[pl.ds(c * EROWS + base, IB)], gidx)
            pltpu.sync_copy(dst_hbm.at[pl.ds(base, IB)], didx)

            bufs = (rowsA, rowsB)
            gsems = (semGA, semGB)
            asems = (semSA, semSB)
            gh = [None] * IB
            ah = [None] * IB
            for kk in range(IB):
                if kk >= 2:
                    ah[kk - 2].wait()
                gh[kk] = pltpu.async_copy(x_hbm.at[gidx.at[kk]],
                                          bufs[kk % 2], gsems[kk % 2])
                if kk >= 1:
                    gh[kk - 1].wait()
                    ah[kk - 1] = pltpu.async_copy(
                        bufs[(kk - 1) % 2], acc.at[didx.at[kk - 1]],
                        asems[(kk - 1) % 2], add=True)
            gh[IB - 1].wait()
            ah[IB - 1] = pltpu.async_copy(
                bufs[(IB - 1) % 2], acc.at[didx.at[IB - 1]],
                asems[(IB - 1) % 2], add=True)
            ah[IB - 2].wait()
            ah[IB - 1].wait()

        plsc.subcore_barrier()
        out_base = c * NPAD + s * ROWS_PER_TILE
        pltpu.sync_copy(acc.at[pl.ds(s * ROWS_PER_TILE, ROWS_PER_TILE)],
                        out_hbm.at[pl.ds(out_base, ROWS_PER_TILE)])

    return k(x_stack, srcg, dst)


_SQRT_HALF = 0.7071067811865476
_INV_SQRT2 = 0.7071067811865476


def _gelu_exact(h):
    return 0.5 * h * (1.0 + lax.erf(h * _SQRT_HALF))


def _tc_mlp(x, agg_lo, agg_hi, W1, b1, W2, b2, eps_i):
    def body(x_ref, lo_ref, hi_ref, w1_ref, b1_ref, w2_ref, b2_ref, e_ref,
             z_ref, s_ref, ss_ref):
        i = pl.program_id(0)
        agg = jnp.concatenate([lo_ref[...], hi_ref[...]], axis=1)
        h0 = (1.0 + e_ref[0, 0]) * x_ref[...] + agg
        h1 = jnp.dot(h0, w1_ref[...], preferred_element_type=jnp.float32,
                     precision=lax.Precision.HIGHEST) + b1_ref[...]
        h1 = _gelu_exact(h1)
        z = jnp.dot(h1, w2_ref[...], preferred_element_type=jnp.float32,
                    precision=lax.Precision.HIGHEST) + b2_ref[...]
        z_ref[...] = z
        rowid = lax.broadcasted_iota(jnp.int32, (8, D), 0)
        pad_s = jnp.where(rowid == 0, jnp.sum(z, axis=0, keepdims=True), 0.0)
        pad_ss = jnp.where(rowid == 0, jnp.sum(z * z, axis=0, keepdims=True), 0.0)

        @pl.when(i == 0)
        def _():
            s_ref[...] = pad_s
            ss_ref[...] = pad_ss

        @pl.when(i > 0)
        def _():
            s_ref[...] = s_ref[...] + pad_s
            ss_ref[...] = ss_ref[...] + pad_ss

    return pl.pallas_call(
        body,
        grid=(NBLK,),
        in_specs=[
            pl.BlockSpec((BN, D), lambda i: (i, 0)),
            pl.BlockSpec((BN, HALF), lambda i: (i, 0)),
            pl.BlockSpec((BN, HALF), lambda i: (i, 0)),
            pl.BlockSpec((D, D), lambda i: (0, 0)),
            pl.BlockSpec((1, D), lambda i: (0, 0)),
            pl.BlockSpec((D, D), lambda i: (0, 0)),
            pl.BlockSpec((1, D), lambda i: (0, 0)),
            pl.BlockSpec((1, 1), lambda i: (0, 0)),
        ],
        out_specs=[
            pl.BlockSpec((BN, D), lambda i: (i, 0)),
            pl.BlockSpec((8, D), lambda i: (0, 0)),
            pl.BlockSpec((8, D), lambda i: (0, 0)),
        ],
        out_shape=[
            jax.ShapeDtypeStruct((N, D), jnp.float32),
            jax.ShapeDtypeStruct((8, D), jnp.float32),
            jax.ShapeDtypeStruct((8, D), jnp.float32),
        ],
    )(x, agg_lo, agg_hi, W1, b1.reshape(1, D), W2, b2.reshape(1, D),
      eps_i.reshape(1, 1))


def _tc_norm(z, x, ssum, ssq, gamma_i, beta_i):
    def body(z_ref, x_ref, s_ref, ss_ref, g_ref, b_ref, o_ref):
        ssum_v = jnp.sum(s_ref[...], axis=0, keepdims=True)
        ssq_v = jnp.sum(ss_ref[...], axis=0, keepdims=True)
        mean = ssum_v * (1.0 / N)
        var = ssq_v * (1.0 / N) - mean * mean
        inv = lax.rsqrt(var + 1e-5)
        h = (z_ref[...] - mean) * (inv * g_ref[...]) + b_ref[...]
        h = _gelu_exact(h)
        o_ref[...] = (x_ref[...] + h) * _INV_SQRT2

    return pl.pallas_call(
        body,
        grid=(NBLK,),
        in_specs=[
            pl.BlockSpec((BN, D), lambda i: (i, 0)),
            pl.BlockSpec((BN, D), lambda i: (i, 0)),
            pl.BlockSpec((8, D), lambda i: (0, 0)),
            pl.BlockSpec((8, D), lambda i: (0, 0)),
            pl.BlockSpec((1, D), lambda i: (0, 0)),
            pl.BlockSpec((1, D), lambda i: (0, 0)),
        ],
        out_specs=pl.BlockSpec((BN, D), lambda i: (i, 0)),
        out_shape=jax.ShapeDtypeStruct((N, D), jnp.float32),
    )(z, x, ssum, ssq, gamma_i.reshape(1, D), beta_i.reshape(1, D))


def kernel(x, edge_index, edge_attr, W1s, b1s, W2s, b2s, eps, gamma, beta):
    pad = EPAD - E
    src = jnp.concatenate(
        [edge_index[0].astype(jnp.int32), jnp.zeros((pad,), jnp.int32)]
    ).reshape(EROWS, CHUNK)
    srcg = jnp.concatenate([src, src + N], axis=0)
    dst = jnp.concatenate(
        [edge_index[1].astype(jnp.int32), jnp.full((pad,), N, jnp.int32)]
    ).reshape(EROWS, CHUNK)

    for i in range(N_LAYERS_K):
        x_stack = jnp.concatenate([x[:, :HALF], x[:, HALF:]], axis=0)
        agg2 = _sc_agg(x_stack, srcg, dst)
        agg_lo = agg2[:N]
        agg_hi = agg2[NPAD:NPAD + N]
        z, ssum, ssq = _tc_mlp(x, agg_lo, agg_hi, W1s[i], b1s[i], W2s[i],
                               b2s[i], eps[i])
        x = _tc_norm(z, x, ssum, ssq, gamma[i], beta[i])
    return x


# speedup vs baseline: 1.2010x; 1.0775x over previous
import functools

import jax
import jax.numpy as jnp
from jax import lax
from jax.experimental import pallas as pl
from jax.experimental.pallas import tpu as pltpu
from jax.experimental.pallas import tpu_sc as plsc

N = 10000
E = 160000
D = 256
HALF = 128
N_LAYERS_K = 3
NS = 16
CHUNK = 128
EPAD = 163840
ED_PER_TILE = EPAD // NS
NCHUNK = ED_PER_TILE // CHUNK
EROWS = EPAD // CHUNK
NPAD = 10240
ROWS_PER_TILE = NPAD // NS

BN = 1000
NBLK = N // BN

IB = 8


def _sc_agg(x, src, dst):
    mesh = plsc.VectorSubcoreMesh(core_axis_name="c", subcore_axis_name="s")

    @functools.partial(
        pl.kernel,
        out_type=jax.ShapeDtypeStruct((NPAD, D), jnp.float32),
        mesh=mesh,
        scratch_types=[
            pltpu.VMEM_SHARED((NPAD, HALF), jnp.float32),
            pltpu.VMEM((IB, CHUNK), jnp.int32),
            pltpu.VMEM((IB, CHUNK), jnp.int32),
            pltpu.VMEM((CHUNK, HALF), jnp.float32),
            pltpu.VMEM((CHUNK, HALF), jnp.float32),
            pltpu.SemaphoreType.DMA,
            pltpu.SemaphoreType.DMA,
            pltpu.SemaphoreType.DMA,
            pltpu.SemaphoreType.DMA,
        ],
    )
    def k(x_hbm, src_hbm, dst_hbm, out_hbm, acc, gidx, didx, rowsA, rowsB,
          semGA, semGB, semSA, semSB):
        c = lax.axis_index("c")
        s = lax.axis_index("s")

        @pl.loop(0, CHUNK)
        def _(r):
            for cc in range(HALF // 16):
                rowsA[r, pl.ds(cc * 16, 16)] = jnp.zeros((16,), jnp.float32)

        @pl.loop(0, ROWS_PER_TILE, step=CHUNK)
        def _(r):
            pltpu.sync_copy(rowsA, acc.at[pl.ds(s * ROWS_PER_TILE + r, CHUNK)])

        plsc.subcore_barrier()

        @pl.loop(0, NCHUNK // IB)
        def _(b):
            base = pl.multiple_of(s * NCHUNK, IB) + b * IB
            pltpu.sync_copy(src_hbm.at[pl.ds(base, IB)], gidx)
            pltpu.sync_copy(dst_hbm.at[pl.ds(base, IB)], didx)

            bufs = (rowsA, rowsB)
            gsems = (semGA, semGB)
            asems = (semSA, semSB)
            gh = [None] * IB
            ah = [None] * IB
            for kk in range(IB):
                if kk >= 2:
                    ah[kk - 2].wait()
                gh[kk] = pltpu.async_copy(
                    x_hbm.at[gidx.at[kk], pl.ds(c * HALF, HALF)],
                    bufs[kk % 2], gsems[kk % 2])
                if kk >= 1:
                    gh[kk - 1].wait()
                    ah[kk - 1] = pltpu.async_copy(
                        bufs[(kk - 1) % 2], acc.at[didx.at[kk - 1]],
                        asems[(kk - 1) % 2], add=True)
            gh[IB - 1].wait()
            ah[IB - 1] = pltpu.async_copy(
                bufs[(IB - 1) % 2], acc.at[didx.at[IB - 1]],
                asems[(IB - 1) % 2], add=True)
            ah[IB - 2].wait()
            ah[IB - 1].wait()

        plsc.subcore_barrier()
        pltpu.sync_copy(acc.at[pl.ds(s * ROWS_PER_TILE, ROWS_PER_TILE)],
                        out_hbm.at[pl.ds(s * ROWS_PER_TILE, ROWS_PER_TILE),
                                   pl.ds(c * HALF, HALF)])

    return k(x, src, dst)


_SQRT_HALF = 0.7071067811865476
_INV_SQRT2 = 0.7071067811865476


def _gelu_exact(h):
    return 0.5 * h * (1.0 + lax.erf(h * _SQRT_HALF))


def _tc_mlp(x, agg, W1, b1, W2, b2, eps_i):
    def body(x_ref, agg_ref, w1_ref, b1_ref, w2_ref, b2_ref, e_ref,
             z_ref, s_ref, ss_ref):
        i = pl.program_id(0)
        h0 = (1.0 + e_ref[0, 0]) * x_ref[...] + agg_ref[...]
        h1 = jnp.dot(h0, w1_ref[...], preferred_element_type=jnp.float32,
                     precision=lax.Precision.HIGHEST) + b1_ref[...]
        h1 = _gelu_exact(h1)
        z = jnp.dot(h1, w2_ref[...], preferred_element_type=jnp.float32,
                    precision=lax.Precision.HIGHEST) + b2_ref[...]
        z_ref[...] = z
        rowid = lax.broadcasted_iota(jnp.int32, (8, D), 0)
        pad_s = jnp.where(rowid == 0, jnp.sum(z, axis=0, keepdims=True), 0.0)
        pad_ss = jnp.where(rowid == 0, jnp.sum(z * z, axis=0, keepdims=True), 0.0)

        @pl.when(i == 0)
        def _():
            s_ref[...] = pad_s
            ss_ref[...] = pad_ss

        @pl.when(i > 0)
        def _():
            s_ref[...] = s_ref[...] + pad_s
            ss_ref[...] = ss_ref[...] + pad_ss

    return pl.pallas_call(
        body,
        grid=(NBLK,),
        in_specs=[
            pl.BlockSpec((BN, D), lambda i: (i, 0)),
            pl.BlockSpec((BN, D), lambda i: (i, 0)),
            pl.BlockSpec((D, D), lambda i: (0, 0)),
            pl.BlockSpec((1, D), lambda i: (0, 0)),
            pl.BlockSpec((D, D), lambda i: (0, 0)),
            pl.BlockSpec((1, D), lambda i: (0, 0)),
            pl.BlockSpec((1, 1), lambda i: (0, 0)),
        ],
        out_specs=[
            pl.BlockSpec((BN, D), lambda i: (i, 0)),
            pl.BlockSpec((8, D), lambda i: (0, 0)),
            pl.BlockSpec((8, D), lambda i: (0, 0)),
        ],
        out_shape=[
            jax.ShapeDtypeStruct((N, D), jnp.float32),
            jax.ShapeDtypeStruct((8, D), jnp.float32),
            jax.ShapeDtypeStruct((8, D), jnp.float32),
        ],
    )(x, agg, W1, b1.reshape(1, D), W2, b2.reshape(1, D), eps_i.reshape(1, 1))


def _tc_norm(z, x, ssum, ssq, gamma_i, beta_i):
    def body(z_ref, x_ref, s_ref, ss_ref, g_ref, b_ref, o_ref):
        ssum_v = jnp.sum(s_ref[...], axis=0, keepdims=True)
        ssq_v = jnp.sum(ss_ref[...], axis=0, keepdims=True)
        mean = ssum_v * (1.0 / N)
        var = ssq_v * (1.0 / N) - mean * mean
        inv = lax.rsqrt(var + 1e-5)
        h = (z_ref[...] - mean) * (inv * g_ref[...]) + b_ref[...]
        h = _gelu_exact(h)
        o_ref[...] = (x_ref[...] + h) * _INV_SQRT2

    return pl.pallas_call(
        body,
        grid=(NBLK,),
        in_specs=[
            pl.BlockSpec((BN, D), lambda i: (i, 0)),
            pl.BlockSpec((BN, D), lambda i: (i, 0)),
            pl.BlockSpec((8, D), lambda i: (0, 0)),
            pl.BlockSpec((8, D), lambda i: (0, 0)),
            pl.BlockSpec((1, D), lambda i: (0, 0)),
            pl.BlockSpec((1, D), lambda i: (0, 0)),
        ],
        out_specs=pl.BlockSpec((BN, D), lambda i: (i, 0)),
        out_shape=jax.ShapeDtypeStruct((N, D), jnp.float32),
    )(z, x, ssum, ssq, gamma_i.reshape(1, D), beta_i.reshape(1, D))


def kernel(x, edge_index, edge_attr, W1s, b1s, W2s, b2s, eps, gamma, beta):
    pad = EPAD - E
    src = jnp.concatenate(
        [edge_index[0].astype(jnp.int32), jnp.zeros((pad,), jnp.int32)]
    ).reshape(EROWS, CHUNK)
    dst = jnp.concatenate(
        [edge_index[1].astype(jnp.int32), jnp.full((pad,), N, jnp.int32)]
    ).reshape(EROWS, CHUNK)

    for i in range(N_LAYERS_K):
        agg = _sc_agg(x, src, dst)
        z, ssum, ssq = _tc_mlp(x, agg, W1s[i], b1s[i], W2s[i], b2s[i], eps[i])
        x = _tc_norm(z, x, ssum, ssq, gamma[i], beta[i])
    return x


# speedup vs baseline: 1.2010x; 1.0000x over previous
import functools

import jax
import jax.numpy as jnp
from jax import lax
from jax.experimental import pallas as pl
from jax.experimental.pallas import tpu as pltpu
from jax.experimental.pallas import tpu_sc as plsc

N = 10000
E = 160000
D = 256
HALF = 128
N_LAYERS_K = 3
NS = 16
CHUNK = 128
EPAD = 163840
ED_PER_TILE = EPAD // NS
NCHUNK = ED_PER_TILE // CHUNK
EROWS = EPAD // CHUNK
NPAD = 10240
ROWS_PER_TILE = NPAD // NS

BN = 1000
NBLK = N // BN

IB = 8


def _sc_agg(x, src, dst):
    mesh = plsc.VectorSubcoreMesh(core_axis_name="c", subcore_axis_name="s")

    @functools.partial(
        pl.kernel,
        out_type=jax.ShapeDtypeStruct((NPAD, D), jnp.float32),
        mesh=mesh,
        scratch_types=[
            pltpu.VMEM_SHARED((NPAD, HALF), jnp.float32),
            pltpu.VMEM((IB, CHUNK), jnp.int32),
            pltpu.VMEM((IB, CHUNK), jnp.int32),
            pltpu.VMEM((CHUNK, HALF), jnp.float32),
            pltpu.VMEM((CHUNK, HALF), jnp.float32),
            pltpu.SemaphoreType.DMA,
            pltpu.SemaphoreType.DMA,
            pltpu.SemaphoreType.DMA,
            pltpu.SemaphoreType.DMA,
        ],
    )
    def k(x_hbm, src_hbm, dst_hbm, out_hbm, acc, gidx, didx, rowsA, rowsB,
          semGA, semGB, semSA, semSB):
        c = lax.axis_index("c")
        s = lax.axis_index("s")

        @pl.loop(0, CHUNK)
        def _(r):
            for cc in range(HALF // 16):
                rowsA[r, pl.ds(cc * 16, 16)] = jnp.zeros((16,), jnp.float32)

        zh = [
            pltpu.async_copy(
                rowsA, acc.at[pl.ds(s * ROWS_PER_TILE + t * CHUNK, CHUNK)],
                semGA)
            for t in range(ROWS_PER_TILE // CHUNK)
        ]
        for h in zh:
            h.wait()

        plsc.subcore_barrier()

        @pl.loop(0, NCHUNK // IB)
        def _(b):
            base = pl.multiple_of(s * NCHUNK, IB) + b * IB
            pltpu.sync_copy(src_hbm.at[pl.ds(base, IB)], gidx)
            pltpu.sync_copy(dst_hbm.at[pl.ds(base, IB)], didx)

            bufs = (rowsA, rowsB)
            gsems = (semGA, semGB)
            asems = (semSA, semSB)
            gh = [None] * IB
            ah = [None] * IB
            for kk in range(IB):
                if kk >= 2:
                    ah[kk - 2].wait()
                gh[kk] = pltpu.async_copy(
                    x_hbm.at[gidx.at[kk], pl.ds(c * HALF, HALF)],
                    bufs[kk % 2], gsems[kk % 2])
                if kk >= 1:
                    gh[kk - 1].wait()
                    ah[kk - 1] = pltpu.async_copy(
                        bufs[(kk - 1) % 2], acc.at[didx.at[kk - 1]],
                        asems[(kk - 1) % 2], add=True)
            gh[IB - 1].wait()
            ah[IB - 1] = pltpu.async_copy(
                bufs[(IB - 1) % 2], acc.at[didx.at[IB - 1]],
                asems[(IB - 1) % 2], add=True)
            ah[IB - 2].wait()
            ah[IB - 1].wait()

        plsc.subcore_barrier()
        pltpu.sync_copy(acc.at[pl.ds(s * ROWS_PER_TILE, ROWS_PER_TILE)],
                        out_hbm.at[pl.ds(s * ROWS_PER_TILE, ROWS_PER_TILE),
                                   pl.ds(c * HALF, HALF)])

    return k(x, src, dst)


_SQRT_HALF = 0.7071067811865476
_INV_SQRT2 = 0.7071067811865476


def _gelu_exact(h):
    return 0.5 * h * (1.0 + lax.erf(h * _SQRT_HALF))


def _tc_mlp(x, agg, W1, b1, W2, b2, eps_i):
    def body(x_ref, agg_ref, w1_ref, b1_ref, w2_ref, b2_ref, e_ref,
             z_ref, s_ref, ss_ref):
        i = pl.program_id(0)
        h0 = (1.0 + e_ref[0, 0]) * x_ref[...] + agg_ref[...]
        h1 = jnp.dot(h0, w1_ref[...], preferred_element_type=jnp.float32,
                     precision=lax.Precision.HIGHEST) + b1_ref[...]
        h1 = _gelu_exact(h1)
        z = jnp.dot(h1, w2_ref[...], preferred_element_type=jnp.float32,
                    precision=lax.Precision.HIGHEST) + b2_ref[...]
        z_ref[...] = z
        rowid = lax.broadcasted_iota(jnp.int32, (8, D), 0)
        pad_s = jnp.where(rowid == 0, jnp.sum(z, axis=0, keepdims=True), 0.0)
        pad_ss = jnp.where(rowid == 0, jnp.sum(z * z, axis=0, keepdims=True), 0.0)

        @pl.when(i == 0)
        def _():
            s_ref[...] = pad_s
            ss_ref[...] = pad_ss

        @pl.when(i > 0)
        def _():
            s_ref[...] = s_ref[...] + pad_s
            ss_ref[...] = ss_ref[...] + pad_ss

    return pl.pallas_call(
        body,
        grid=(NBLK,),
        in_specs=[
            pl.BlockSpec((BN, D), lambda i: (i, 0)),
            pl.BlockSpec((BN, D), lambda i: (i, 0)),
            pl.BlockSpec((D, D), lambda i: (0, 0)),
            pl.BlockSpec((1, D), lambda i: (0, 0)),
            pl.BlockSpec((D, D), lambda i: (0, 0)),
            pl.BlockSpec((1, D), lambda i: (0, 0)),
            pl.BlockSpec((1, 1), lambda i: (0, 0)),
        ],
        out_specs=[
            pl.BlockSpec((BN, D), lambda i: (i, 0)),
            pl.BlockSpec((8, D), lambda i: (0, 0)),
            pl.BlockSpec((8, D), lambda i: (0, 0)),
        ],
        out_shape=[
            jax.ShapeDtypeStruct((N, D), jnp.float32),
            jax.ShapeDtypeStruct((8, D), jnp.float32),
            jax.ShapeDtypeStruct((8, D), jnp.float32),
        ],
    )(x, agg, W1, b1.reshape(1, D), W2, b2.reshape(1, D), eps_i.reshape(1, 1))


def _tc_norm(z, x, ssum, ssq, gamma_i, beta_i):
    def body(z_ref, x_ref, s_ref, ss_ref, g_ref, b_ref, o_ref):
        ssum_v = jnp.sum(s_ref[...], axis=0, keepdims=True)
        ssq_v = jnp.sum(ss_ref[...], axis=0, keepdims=True)
        mean = ssum_v * (1.0 / N)
        var = ssq_v * (1.0 / N) - mean * mean
        inv = lax.rsqrt(var + 1e-5)
        h = (z_ref[...] - mean) * (inv * g_ref[...]) + b_ref[...]
        h = _gelu_exact(h)
        o_ref[...] = (x_ref[...] + h) * _INV_SQRT2

    return pl.pallas_call(
        body,
        grid=(NBLK,),
        in_specs=[
            pl.BlockSpec((BN, D), lambda i: (i, 0)),
            pl.BlockSpec((BN, D), lambda i: (i, 0)),
            pl.BlockSpec((8, D), lambda i: (0, 0)),
            pl.BlockSpec((8, D), lambda i: (0, 0)),
            pl.BlockSpec((1, D), lambda i: (0, 0)),
            pl.BlockSpec((1, D), lambda i: (0, 0)),
        ],
        out_specs=pl.BlockSpec((BN, D), lambda i: (i, 0)),
        out_shape=jax.ShapeDtypeStruct((N, D), jnp.float32),
    )(z, x, ssum, ssq, gamma_i.reshape(1, D), beta_i.reshape(1, D))


def kernel(x, edge_index, edge_attr, W1s, b1s, W2s, b2s, eps, gamma, beta):
    pad = EPAD - E
    src = jnp.concatenate(
        [edge_index[0].astype(jnp.int32), jnp.zeros((pad,), jnp.int32)]
    ).reshape(EROWS, CHUNK)
    dst = jnp.concatenate(
        [edge_index[1].astype(jnp.int32), jnp.full((pad,), N, jnp.int32)]
    ).reshape(EROWS, CHUNK)

    for i in range(N_LAYERS_K):
        agg = _sc_agg(x, src, dst)
        z, ssum, ssq = _tc_mlp(x, agg, W1s[i], b1s[i], W2s[i], b2s[i], eps[i])
        x = _tc_norm(z, x, ssum, ssq, gamma[i], beta[i])
    return x
